# baseline (speedup 1.0000x reference)
import functools

import jax
import jax.numpy as jnp
from jax import lax
from jax.experimental import pallas as pl
from jax.experimental.pallas import tpu as pltpu
from jax.experimental.pallas import tpu_sc as plsc

E = 8
G = 2
EG = E // G
D = 768
F = 512
T = 2048
B = 256
NB_G = T // B + EG
NBT = G * NB_G
PG = NB_G * B
ROWS = G * PG

NC, NS = 2, 16
NW = NC * NS


def _routing_body(x_ref, rw_ref, out_ref):
    x = x_ref[...]
    logits = jnp.dot(x, rw_ref[...], preferred_element_type=jnp.float32)
    m = jnp.max(logits, axis=-1, keepdims=True)
    ex = jnp.exp(logits - m)
    p = ex / jnp.sum(ex, axis=-1, keepdims=True)
    col = lax.broadcasted_iota(jnp.int32, (T, E), 1)
    g0 = col < EG
    neg = jnp.float32(-1.0)
    w0 = jnp.max(jnp.where(g0, p, neg), axis=-1, keepdims=True)
    w1 = jnp.max(jnp.where(g0, neg, p), axis=-1, keepdims=True)
    big = jnp.int32(E)
    e0 = jnp.min(jnp.where(g0 & (p == w0), col, big), axis=-1, keepdims=True)
    e1 = jnp.min(jnp.where((~g0) & (p == w1), col, big), axis=-1, keepdims=True)
    s = w0 + w1
    w0n = w0 / s
    w1n = w1 / s
    oc = lax.broadcasted_iota(jnp.int32, (T, EG), 1)
    out = jnp.where(oc == 0, e0.astype(jnp.float32),
          jnp.where(oc == 1, e1.astype(jnp.float32),
          jnp.where(oc == 2, w0n, w1n)))
    out_ref[...] = out


def _routing(x, router_w):
    return pl.pallas_call(
        _routing_body,
        out_shape=jax.ShapeDtypeStruct((T, EG), jnp.float32),
    )(x, router_w)


def _make_mesh():
    return plsc.VectorSubcoreMesh(
        core_axis_name="c", subcore_axis_name="s",
        num_cores=NC, num_subcores=NS)


_GCH = 64
_RPW = ROWS // NW


def _gather_body(x_hbm, idx_hbm, out_hbm, idx_v, rows_v, sem):
    wid = lax.axis_index("s") * NC + lax.axis_index("c")
    base = wid * _RPW
    for ci in range(_RPW // _GCH):
        off = base + ci * _GCH
        pltpu.sync_copy(idx_hbm.at[pl.ds(off, _GCH)], idx_v)
        pltpu.async_copy(x_hbm.at[idx_v], rows_v, sem).wait()
        pltpu.sync_copy(rows_v, out_hbm.at[pl.ds(off, _GCH)])


def _dispatch(x, tok_sorted):
    return pl.kernel(
        _gather_body,
        out_type=jax.ShapeDtypeStruct((ROWS, D), jnp.float32),
        mesh=_make_mesh(),
        scratch_types=[
            pltpu.VMEM((_GCH,), jnp.int32),
            pltpu.VMEM((_GCH, D), jnp.float32),
            pltpu.SemaphoreType.DMA,
        ],
    )(x, tok_sorted)


def _ffn_body(be_ref, x_ref, wgu_ref, wd_ref, ws_ref, y_ref):
    del be_ref
    x = x_ref[...]
    gu = jnp.dot(x, wgu_ref[0], preferred_element_type=jnp.float32)
    g = gu[:, :F]
    u = gu[:, F:]
    h = g * (1.0 / (1.0 + jnp.exp(-g))) * u
    h = h * ws_ref[...]
    y_ref[...] = jnp.dot(h, wd_ref[0], preferred_element_type=jnp.float32)


def _ffn(block_expert, x_sorted, w_gate_up, w_down, w_sorted):
    grid_spec = pltpu.PrefetchScalarGridSpec(
        num_scalar_prefetch=1,
        grid=(NBT,),
        in_specs=[
            pl.BlockSpec((B, D), lambda b, be: (b, 0)),
            pl.BlockSpec((1, D, 2 * F), lambda b, be: (be[b], 0, 0)),
            pl.BlockSpec((1, F, D), lambda b, be: (be[b], 0, 0)),
            pl.BlockSpec((B, 1), lambda b, be: (b, 0)),
        ],
        out_specs=pl.BlockSpec((B, D), lambda b, be: (b, 0)),
    )
    return pl.pallas_call(
        _ffn_body,
        grid_spec=grid_spec,
        out_shape=jax.ShapeDtypeStruct((ROWS, D), jnp.float32),
    )(block_expert, x_sorted, w_gate_up, w_down, w_sorted)


_CPW = T // NW


def _combine_body(y_hbm, p0_hbm, p1_hbm, out_hbm, i0_v, i1_v, b0_v, b1_v, sem):
    wid = lax.axis_index("s") * NC + lax.axis_index("c")
    base = wid * _CPW
    pltpu.sync_copy(p0_hbm.at[pl.ds(base, _CPW)], i0_v)
    pltpu.sync_copy(p1_hbm.at[pl.ds(base, _CPW)], i1_v)
    pltpu.async_copy(y_hbm.at[i0_v], b0_v, sem).wait()
    pltpu.async_copy(y_hbm.at[i1_v], b1_v, sem).wait()

    def row_add(r, carry):
        for j in range(D // 16):
            sl = (r, pl.ds(j * 16, 16))
            b0_v[sl] = b0_v[sl] + b1_v[sl]
        return carry

    lax.fori_loop(0, _CPW, row_add, 0)
    pltpu.sync_copy(b0_v, out_hbm.at[pl.ds(base, _CPW)])


def _combine(y_sorted, pos0, pos1):
    return pl.kernel(
        _combine_body,
        out_type=jax.ShapeDtypeStruct((T, D), jnp.float32),
        mesh=_make_mesh(),
        scratch_types=[
            pltpu.VMEM((_CPW,), jnp.int32),
            pltpu.VMEM((_CPW,), jnp.int32),
            pltpu.VMEM((_CPW, D), jnp.float32),
            pltpu.VMEM((_CPW, D), jnp.float32),
            pltpu.SemaphoreType.DMA,
        ],
    )(y_sorted, pos0, pos1)


def _positions(eid, w):
    onehot = (eid[:, None] == jnp.arange(EG, dtype=jnp.int32)[None, :])
    oh = onehot.astype(jnp.int32)
    counts = jnp.sum(oh, axis=0)
    rank = jnp.take_along_axis(jnp.cumsum(oh, axis=0) - oh,
                               eid[:, None], axis=1)[:, 0]
    pc = ((counts + B - 1) // B) * B
    cum = jnp.cumsum(pc)
    off = cum - pc
    pos = off[eid] + rank
    starts = jnp.arange(NB_G, dtype=jnp.int32) * B
    block_expert = jnp.clip(
        jnp.searchsorted(cum, starts, side="right").astype(jnp.int32),
        0, EG - 1)
    tok = jnp.zeros((PG,), jnp.int32).at[pos].set(
        jnp.arange(T, dtype=jnp.int32))
    ws = jnp.zeros((PG,), jnp.float32).at[pos].set(w)
    return pos, tok, ws, block_expert


def kernel(hidden_states, router_w, w_gate_up, w_down):
    x = hidden_states
    routed = _routing(x, router_w)
    e0 = routed[:, 0].astype(jnp.int32)
    e1 = routed[:, 1].astype(jnp.int32) - EG
    w0 = routed[:, 2]
    w1 = routed[:, 3]

    pos0, tok0, ws0, be0 = _positions(e0, w0)
    pos1, tok1, ws1, be1 = _positions(e1, w1)
    tok_sorted = jnp.concatenate([tok0, tok1])
    w_sorted = jnp.concatenate([ws0, ws1])[:, None]
    block_expert = jnp.concatenate([be0, be1 + EG])

    x_sorted = _dispatch(x, tok_sorted)
    y_sorted = _ffn(block_expert, x_sorted, w_gate_up, w_down, w_sorted)
    out = _combine(y_sorted, pos0, pos1 + PG)
    return out.astype(hidden_states.dtype)


# speedup vs baseline: 1.6107x; 1.6107x over previous
import functools

import jax
import jax.numpy as jnp
from jax import lax
from jax.experimental import pallas as pl
from jax.experimental.pallas import tpu as pltpu
from jax.experimental.pallas import tpu_sc as plsc

E = 8
G = 2
EG = E // G
D = 768
F = 512
T = 2048
B = 256
NB_G = T // B + EG
NBT = G * NB_G
PG = NB_G * B
ROWS = G * PG

NC, NS = 2, 16
NW = NC * NS


def _routing_body(x_ref, rw_ref, out_ref):
    x = x_ref[...]
    logits = jnp.dot(x, rw_ref[...], preferred_element_type=jnp.float32)
    m = jnp.max(logits, axis=-1, keepdims=True)
    ex = jnp.exp(logits - m)
    p = ex / jnp.sum(ex, axis=-1, keepdims=True)
    col = lax.broadcasted_iota(jnp.int32, (T, E), 1)
    g0 = col < EG
    neg = jnp.float32(-1.0)
    w0 = jnp.max(jnp.where(g0, p, neg), axis=-1, keepdims=True)
    w1 = jnp.max(jnp.where(g0, neg, p), axis=-1, keepdims=True)
    big = jnp.int32(E)
    e0 = jnp.min(jnp.where(g0 & (p == w0), col, big), axis=-1, keepdims=True)
    e1 = jnp.min(jnp.where((~g0) & (p == w1), col, big), axis=-1, keepdims=True)
    s = w0 + w1
    w0n = w0 / s
    w1n = w1 / s
    oc = lax.broadcasted_iota(jnp.int32, (T, EG), 1)
    out = jnp.where(oc == 0, e0.astype(jnp.float32),
          jnp.where(oc == 1, e1.astype(jnp.float32),
          jnp.where(oc == 2, w0n, w1n)))
    out_ref[...] = out


def _routing(x, router_w):
    return pl.pallas_call(
        _routing_body,
        out_shape=jax.ShapeDtypeStruct((T, EG), jnp.float32),
    )(x, router_w)


def _make_mesh():
    return plsc.VectorSubcoreMesh(
        core_axis_name="c", subcore_axis_name="s",
        num_cores=NC, num_subcores=NS)


_GCH = 64
_RPW = ROWS // NW


def _gather_body(x_hbm, idx_hbm, out_hbm, idx_v, rows0_v, rows1_v,
                 gsem0, gsem1, wsem0, wsem1):
    wid = lax.axis_index("s") * NC + lax.axis_index("c")
    base = wid * _RPW
    pltpu.sync_copy(idx_hbm.at[pl.ds(base, _RPW)], idx_v)
    nch = _RPW // _GCH
    bufs = (rows0_v, rows1_v)
    gsems = (gsem0, gsem1)
    wsems = (wsem0, wsem1)
    g = []
    w = [None, None]
    for ci in range(nch):
        s = ci % 2
        if ci >= 2:
            w[s].wait()
        g.append(pltpu.async_copy(
            x_hbm.at[idx_v.at[pl.ds(ci * _GCH, _GCH)]], bufs[s], gsems[s]))
        g[ci].wait()
        w[s] = pltpu.async_copy(
            bufs[s], out_hbm.at[pl.ds(base + ci * _GCH, _GCH)], wsems[s])
    for s in range(2):
        if w[s] is not None:
            w[s].wait()


def _dispatch(x, tok_sorted):
    return pl.kernel(
        _gather_body,
        out_type=jax.ShapeDtypeStruct((ROWS, D), jnp.float32),
        mesh=_make_mesh(),
        scratch_types=[
            pltpu.VMEM((_RPW,), jnp.int32),
            pltpu.VMEM((_GCH, D), jnp.float32),
            pltpu.VMEM((_GCH, D), jnp.float32),
            pltpu.SemaphoreType.DMA,
            pltpu.SemaphoreType.DMA,
            pltpu.SemaphoreType.DMA,
            pltpu.SemaphoreType.DMA,
        ],
    )(x, tok_sorted)


def _ffn_body(be_ref, x_ref, wgu_ref, wd_ref, ws_ref, y_ref):
    del be_ref
    x = x_ref[...]
    gu = jnp.dot(x, wgu_ref[0], preferred_element_type=jnp.float32)
    g = gu[:, :F]
    u = gu[:, F:]
    h = g * (1.0 / (1.0 + jnp.exp(-g))) * u
    h = h * ws_ref[...]
    y_ref[...] = jnp.dot(h, wd_ref[0], preferred_element_type=jnp.float32)


def _ffn(block_expert, x_sorted, w_gate_up, w_down, w_sorted):
    grid_spec = pltpu.PrefetchScalarGridSpec(
        num_scalar_prefetch=1,
        grid=(NBT,),
        in_specs=[
            pl.BlockSpec((B, D), lambda b, be: (b, 0)),
            pl.BlockSpec((1, D, 2 * F), lambda b, be: (be[b], 0, 0)),
            pl.BlockSpec((1, F, D), lambda b, be: (be[b], 0, 0)),
            pl.BlockSpec((B, 1), lambda b, be: (b, 0)),
        ],
        out_specs=pl.BlockSpec((B, D), lambda b, be: (b, 0)),
    )
    return pl.pallas_call(
        _ffn_body,
        grid_spec=grid_spec,
        out_shape=jax.ShapeDtypeStruct((ROWS, D), jnp.float32),
    )(block_expert, x_sorted, w_gate_up, w_down, w_sorted)


_CPW = T // NW


def _combine_body(y_hbm, p0_hbm, p1_hbm, out_hbm, i0_v, i1_v, b0_v, b1_v, sem):
    wid = lax.axis_index("s") * NC + lax.axis_index("c")
    base = wid * _CPW
    pltpu.sync_copy(p0_hbm.at[pl.ds(base, _CPW)], i0_v)
    pltpu.sync_copy(p1_hbm.at[pl.ds(base, _CPW)], i1_v)
    pltpu.async_copy(y_hbm.at[i0_v], b0_v, sem).wait()
    pltpu.async_copy(y_hbm.at[i1_v], b1_v, sem).wait()

    def row_add(r, carry):
        for j in range(D // 16):
            sl = (r, pl.ds(j * 16, 16))
            b0_v[sl] = b0_v[sl] + b1_v[sl]
        return carry

    lax.fori_loop(0, _CPW, row_add, 0)
    pltpu.sync_copy(b0_v, out_hbm.at[pl.ds(base, _CPW)])


def _combine(y_sorted, pos0, pos1):
    return pl.kernel(
        _combine_body,
        out_type=jax.ShapeDtypeStruct((T, D), jnp.float32),
        mesh=_make_mesh(),
        scratch_types=[
            pltpu.VMEM((_CPW,), jnp.int32),
            pltpu.VMEM((_CPW,), jnp.int32),
            pltpu.VMEM((_CPW, D), jnp.float32),
            pltpu.VMEM((_CPW, D), jnp.float32),
            pltpu.SemaphoreType.DMA,
        ],
    )(y_sorted, pos0, pos1)


def _positions(eid, w):
    onehot = (eid[:, None] == jnp.arange(EG, dtype=jnp.int32)[None, :])
    oh = onehot.astype(jnp.int32)
    counts = jnp.sum(oh, axis=0)
    rank = jnp.take_along_axis(jnp.cumsum(oh, axis=0) - oh,
                               eid[:, None], axis=1)[:, 0]
    pc = ((counts + B - 1) // B) * B
    cum = jnp.cumsum(pc)
    off = cum - pc
    pos = off[eid] + rank
    starts = jnp.arange(NB_G, dtype=jnp.int32) * B
    block_expert = jnp.clip(
        jnp.searchsorted(cum, starts, side="right").astype(jnp.int32),
        0, EG - 1)
    pad_spread = jnp.arange(PG, dtype=jnp.int32) % T
    tok = pad_spread.at[pos].set(jnp.arange(T, dtype=jnp.int32))
    ws = jnp.zeros((PG,), jnp.float32).at[pos].set(w)
    return pos, tok, ws, block_expert


def kernel(hidden_states, router_w, w_gate_up, w_down):
    x = hidden_states
    routed = _routing(x, router_w)
    e0 = routed[:, 0].astype(jnp.int32)
    e1 = routed[:, 1].astype(jnp.int32) - EG
    w0 = routed[:, 2]
    w1 = routed[:, 3]

    pos0, tok0, ws0, be0 = _positions(e0, w0)
    pos1, tok1, ws1, be1 = _positions(e1, w1)
    tok_sorted = jnp.concatenate([tok0, tok1])
    w_sorted = jnp.concatenate([ws0, ws1])[:, None]
    block_expert = jnp.concatenate([be0, be1 + EG])

    x_sorted = _dispatch(x, tok_sorted)
    y_sorted = _ffn(block_expert, x_sorted, w_gate_up, w_down, w_sorted)
    out = _combine(y_sorted, pos0, pos1 + PG)
    return out.astype(hidden_states.dtype)


# speedup vs baseline: 2.4681x; 1.5323x over previous
import jax
import jax.numpy as jnp
from jax import lax
from jax.experimental import pallas as pl
from jax.experimental.pallas import tpu as pltpu
from jax.experimental.pallas import tpu_sc as plsc

E = 8
G = 2
EG = E // G
D = 768
F = 512
T = 2048
B = 256
NB_G = T // B + EG
NBT = G * NB_G
PG = NB_G * B
ROWS = G * PG
CHUNK = 128

NC, NS = 2, 16
NW = NC * NS
TPW = T // NW


def _routing_body(x_ref, rw_ref, pos_ref, w_ref, cnt_ref):
    x = x_ref[...]
    logits = jnp.dot(x, rw_ref[...], preferred_element_type=jnp.float32)
    m = jnp.max(logits, axis=-1, keepdims=True)
    ex = jnp.exp(logits - m)
    p = ex / jnp.sum(ex, axis=-1, keepdims=True)
    col = lax.broadcasted_iota(jnp.int32, (T, E), 1)
    g0 = col < EG
    neg = jnp.float32(-1.0)
    w0 = jnp.max(jnp.where(g0, p, neg), axis=-1, keepdims=True)
    w1 = jnp.max(jnp.where(g0, neg, p), axis=-1, keepdims=True)
    big = jnp.int32(E)
    e0 = jnp.min(jnp.where(g0 & (p == w0), col, big), axis=-1, keepdims=True)
    e1 = jnp.min(jnp.where((~g0) & (p == w1), col, big), axis=-1, keepdims=True)
    s = w0 + w1
    oh0 = (col == e0).astype(jnp.float32)
    oh1 = (col == e1).astype(jnp.float32)
    oh = oh0 + oh1

    r_ = lax.broadcasted_iota(jnp.int32, (CHUNK, CHUNK), 0)
    c_ = lax.broadcasted_iota(jnp.int32, (CHUNK, CHUNK), 1)
    tril = (r_ >= c_).astype(jnp.float32)
    carry = jnp.zeros((1, E), jnp.float32)
    rank_chunks = []
    for c in range(T // CHUNK):
        oh_c = oh[c * CHUNK:(c + 1) * CHUNK, :]
        cum_c = jnp.dot(tril, oh_c, preferred_element_type=jnp.float32)
        rank_chunks.append(cum_c + carry - oh_c)
        carry = carry + jnp.sum(oh_c, axis=0, keepdims=True)
    rank = jnp.concatenate(rank_chunks, axis=0)

    counts = carry
    pc = (jnp.floor(counts / B) +
          jnp.where(counts % B > 0, 1.0, 0.0)) * B
    gi = lax.broadcasted_iota(jnp.int32, (E, E), 0)
    gj = lax.broadcasted_iota(jnp.int32, (E, E), 1)
    prefix = ((gi // EG == gj // EG) & (gi < gj)).astype(jnp.float32)
    off = jnp.dot(pc, prefix, preferred_element_type=jnp.float32)

    posf = jnp.sum(oh * (off + rank), axis=-1, keepdims=True)
    pos0 = jnp.sum(oh0 * (off + rank), axis=-1, keepdims=True)
    pos1 = posf - pos0 + PG
    pos2 = jnp.concatenate([pos0, pos1], axis=1)
    pos_ref[...] = jnp.transpose(pos2, (1, 0)).astype(jnp.int32)
    w2 = jnp.concatenate([w0 / s, w1 / s], axis=1)
    w_ref[...] = jnp.transpose(w2, (1, 0))
    cnt_ref[...] = counts.astype(jnp.int32)


def _routing(x, router_w):
    return pl.pallas_call(
        _routing_body,
        out_shape=(
            jax.ShapeDtypeStruct((G, T), jnp.int32),
            jax.ShapeDtypeStruct((G, T), jnp.float32),
            jax.ShapeDtypeStruct((1, E), jnp.int32),
        ),
    )(x, router_w)


def _make_mesh():
    return plsc.VectorSubcoreMesh(
        core_axis_name="c", subcore_axis_name="s",
        num_cores=NC, num_subcores=NS)


def _scatter_body(x_hbm, p0_hbm, p1_hbm, out_hbm, i0_v, i1_v, rows_v,
                  sem0, sem1):
    wid = lax.axis_index("s") * NC + lax.axis_index("c")
    base = wid * TPW
    pltpu.sync_copy(p0_hbm.at[pl.ds(base, TPW)], i0_v)
    pltpu.sync_copy(p1_hbm.at[pl.ds(base, TPW)], i1_v)
    pltpu.sync_copy(x_hbm.at[pl.ds(base, TPW)], rows_v)
    c0 = pltpu.async_copy(rows_v, out_hbm.at[i0_v], sem0)
    c1 = pltpu.async_copy(rows_v, out_hbm.at[i1_v], sem1)
    c0.wait()
    c1.wait()


def _dispatch(x, pos0, pos1):
    return pl.kernel(
        _scatter_body,
        out_type=jax.ShapeDtypeStruct((ROWS, D), jnp.float32),
        mesh=_make_mesh(),
        scratch_types=[
            pltpu.VMEM((TPW,), jnp.int32),
            pltpu.VMEM((TPW,), jnp.int32),
            pltpu.VMEM((TPW, D), jnp.float32),
            pltpu.SemaphoreType.DMA,
            pltpu.SemaphoreType.DMA,
        ],
    )(x, pos0, pos1)


def _ffn_body(be_ref, x_ref, wgu_ref, wd_ref, y_ref):
    del be_ref
    x = x_ref[...]
    gu = jnp.dot(x, wgu_ref[0], preferred_element_type=jnp.float32)
    g = gu[:, :F]
    u = gu[:, F:]
    h = g * (1.0 / (1.0 + jnp.exp(-g))) * u
    y_ref[...] = jnp.dot(h, wd_ref[0], preferred_element_type=jnp.float32)


def _ffn(block_expert, x_sorted, w_gate_up, w_down):
    grid_spec = pltpu.PrefetchScalarGridSpec(
        num_scalar_prefetch=1,
        grid=(NBT,),
        in_specs=[
            pl.BlockSpec((B, D), lambda b, be: (b, 0)),
            pl.BlockSpec((1, D, 2 * F), lambda b, be: (be[b], 0, 0)),
            pl.BlockSpec((1, F, D), lambda b, be: (be[b], 0, 0)),
        ],
        out_specs=pl.BlockSpec((B, D), lambda b, be: (b, 0)),
    )
    return pl.pallas_call(
        _ffn_body,
        grid_spec=grid_spec,
        out_shape=jax.ShapeDtypeStruct((ROWS, D), jnp.float32),
    )(block_expert, x_sorted, w_gate_up, w_down)


def _combine_body(y_hbm, p0_hbm, p1_hbm, w0_hbm, w1_hbm, out_hbm,
                  i0_v, i1_v, w0_v, w1_v, b0_v, b1_v, sem):
    wid = lax.axis_index("s") * NC + lax.axis_index("c")
    base = wid * TPW
    pltpu.sync_copy(p0_hbm.at[pl.ds(base, TPW)], i0_v)
    pltpu.sync_copy(p1_hbm.at[pl.ds(base, TPW)], i1_v)
    pltpu.sync_copy(w0_hbm.at[pl.ds(base, TPW)], w0_v)
    pltpu.sync_copy(w1_hbm.at[pl.ds(base, TPW)], w1_v)
    pltpu.async_copy(y_hbm.at[i0_v], b0_v, sem).wait()
    pltpu.async_copy(y_hbm.at[i1_v], b1_v, sem).wait()

    def row_fma(r, carry):
        wa = w0_v[r]
        wb = w1_v[r]
        for j in range(D // 16):
            sl = (r, pl.ds(j * 16, 16))
            b0_v[sl] = b0_v[sl] * wa + b1_v[sl] * wb
        return carry

    lax.fori_loop(0, TPW, row_fma, 0)
    pltpu.sync_copy(b0_v, out_hbm.at[pl.ds(base, TPW)])


def _combine(y_sorted, pos0, pos1, w0, w1):
    return pl.kernel(
        _combine_body,
        out_type=jax.ShapeDtypeStruct((T, D), jnp.float32),
        mesh=_make_mesh(),
        scratch_types=[
            pltpu.VMEM((TPW,), jnp.int32),
            pltpu.VMEM((TPW,), jnp.int32),
            pltpu.VMEM((TPW, 16), jnp.float32),
            pltpu.VMEM((TPW, 16), jnp.float32),
            pltpu.VMEM((TPW, D), jnp.float32),
            pltpu.VMEM((TPW, D), jnp.float32),
            pltpu.SemaphoreType.DMA,
        ],
    )(y_sorted, pos0, pos1, w0, w1)


def kernel(hidden_states, router_w, w_gate_up, w_down):
    x = hidden_states
    pos, w, counts = _routing(x, router_w)
    pos0, pos1 = pos[0], pos[1]
    w0, w1 = w[0], w[1]

    cnt = counts[0]
    pc = ((cnt + B - 1) // B) * B
    starts = (jnp.arange(NB_G, dtype=jnp.int32) * B)[:, None]
    c0 = jnp.cumsum(pc[:EG])
    c1 = jnp.cumsum(pc[EG:])
    be0 = jnp.minimum(jnp.sum((starts >= c0[None, :]).astype(jnp.int32),
                              axis=1), EG - 1)
    be1 = jnp.minimum(jnp.sum((starts >= c1[None, :]).astype(jnp.int32),
                              axis=1), EG - 1)
    block_expert = jnp.concatenate([be0, be1 + EG])

    w0r = jnp.broadcast_to(w0[:, None], (T, 16))
    w1r = jnp.broadcast_to(w1[:, None], (T, 16))

    x_sorted = _dispatch(x, pos0, pos1)
    y_sorted = _ffn(block_expert, x_sorted, w_gate_up, w_down)
    out = _combine(y_sorted, pos0, pos1, w0r, w1r)
    return out.astype(hidden_states.dtype)
